# Initial kernel scaffold; baseline (speedup 1.0000x reference)
import jax
import jax.numpy as jnp
from jax.experimental import pallas as pl


def kernel(x, edge_index, edge_weight, params):
    raise NotImplementedError("write your pallas kernel here")


# speedup vs baseline: 3.2927x; 3.2927x over previous
import functools

import jax
import jax.numpy as jnp
from jax import lax
from jax.experimental import pallas as pl
from jax.experimental.pallas import tpu as pltpu
from jax.experimental.pallas import tpu_sc as plsc

MM_DT = jnp.float32

KT = 3
KG = 3
C = 64

NC, NS, NL = 2, 16, 16
NW = NC * NS
ECAP = 16384
ECH = 128
PADQ = 2 * ECH
EB = 2000
VT = 400

_MESH = dict(core_axis_name="c", subcore_axis_name="s")
_SC_PARAMS = pltpu.CompilerParams(needs_layout_passes=False)
_TC_PARAMS = pltpu.CompilerParams(vmem_limit_bytes=100 * 1024 * 1024)


def _wid():
    return lax.axis_index("s") * NC + lax.axis_index("c")


def _sc_prep(erow, ecol, edge_weight, VP, NVT):
    E = edge_weight.shape[0]
    nblk = E // EB
    assert E % EB == 0 and E % NL == 0

    @functools.partial(
        pl.kernel,
        out_type=(
            jax.ShapeDtypeStruct((NW, ECAP), jnp.int32),
            jax.ShapeDtypeStruct((NW, ECAP), jnp.int32),
            jax.ShapeDtypeStruct((NW, ECAP), jnp.float32),
            jax.ShapeDtypeStruct((NW, NL), jnp.int32),
            jax.ShapeDtypeStruct((VP,), jnp.float32),
        ),
        mesh=plsc.VectorSubcoreMesh(**_MESH),
        compiler_params=_SC_PARAMS,
        scratch_types=[
            pltpu.VMEM((EB,), jnp.int32),
            pltpu.VMEM((EB,), jnp.int32),
            pltpu.VMEM((EB,), jnp.float32),
            pltpu.VMEM((ECAP,), jnp.int32),
            pltpu.VMEM((ECAP,), jnp.int32),
            pltpu.VMEM((ECAP,), jnp.float32),
            pltpu.VMEM((NVT * NL,), jnp.float32),
            pltpu.VMEM((NVT,), jnp.float32),
            pltpu.VMEM((NL,), jnp.int32),
        ],
    )
    def k(er, ec, ew, rows_o, co_o, wm_o, cnt_o, deg_o,
          rbuf, cbuf, wbuf, lrow, lco, lwm, deg16, degv, c16):
        wid = _wid()
        base = wid * NVT
        zi = jnp.zeros((NL,), jnp.int32)
        zf = jnp.zeros((NL,), jnp.float32)

        def zlist(i, _):
            lrow[pl.ds(i * NL, NL)] = zi
            lco[pl.ds(i * NL, NL)] = zi
            lwm[pl.ds(i * NL, NL)] = zf
            return 0
        lax.fori_loop(0, ECAP // NL, zlist, 0)

        def zdeg(i, _):
            deg16[pl.ds(i * NL, NL)] = zf
            return 0
        lax.fori_loop(0, NVT, zdeg, 0)

        lane_off = lax.iota(jnp.int32, NL) * NVT

        def blk(bi, off):
            pltpu.sync_copy(er.at[pl.ds(bi * EB, EB)], rbuf)
            pltpu.sync_copy(ec.at[pl.ds(bi * EB, EB)], cbuf)
            pltpu.sync_copy(ew.at[pl.ds(bi * EB, EB)], wbuf)

            def ch(ki, off):
                r = rbuf[pl.ds(ki * NL, NL)]
                c = cbuf[pl.ds(ki * NL, NL)]
                w = wbuf[pl.ds(ki * NL, NL)]
                nonself = r != c
                crel = c - base
                mc = (crel >= 0) & (crel < NVT)
                cum = plsc.cumsum(mc.astype(jnp.int32))
                pos = off + cum - 1
                plsc.store_scatter(lrow, [pos], r, mask=mc)
                plsc.store_scatter(lco, [pos], crel * 128, mask=mc)
                plsc.store_scatter(lwm, [pos], jnp.where(nonself, w, 0.0),
                                   mask=mc)
                rrel = r - base
                mr = (rrel >= 0) & (rrel < NVT) & nonself
                idx = jnp.where(mr, rrel, 0) + lane_off
                plsc.addupdate_scatter(deg16, [idx], w, mask=mr)
                return off + cum[NL - 1]
            return lax.fori_loop(0, EB // NL, ch, off)

        off = lax.fori_loop(0, nblk, blk, jnp.int32(0))

        for vc in range(NVT // NL):
            acc = zf
            for cp in range(NL):
                acc = acc + deg16[pl.ds(cp * NVT + vc * NL, NL)]
            degv[pl.ds(vc * NL, NL)] = acc

        poff = ((off + PADQ - 1) // PADQ) * PADQ
        c16[...] = jnp.full((NL,), 0, jnp.int32) + poff
        pltpu.sync_copy(c16, cnt_o.at[wid])
        pltpu.sync_copy(lrow, rows_o.at[wid])
        pltpu.sync_copy(lco, co_o.at[wid])
        pltpu.sync_copy(lwm, wm_o.at[wid])
        pltpu.sync_copy(degv, deg_o.at[pl.ds(base, NVT)])

    return k(erow, ecol, edge_weight)


def _tc_dinv(deg, VP):
    def body(d_ref, o_ref):
        d = d_ref[...]
        o_ref[...] = jnp.where(d > 0, lax.rsqrt(jnp.where(d > 0, d, 1.0)),
                               0.0)
    out = pl.pallas_call(
        body,
        out_shape=jax.ShapeDtypeStruct((VP // 128, 128), jnp.float32),
    )(deg.reshape(VP // 128, 128))
    return out.reshape(VP)


def _sc_norm(rows, co, wm, cnt, dinv, VP, NVT):
    @functools.partial(
        pl.kernel,
        out_type=jax.ShapeDtypeStruct((NW, ECAP), jnp.float32),
        mesh=plsc.VectorSubcoreMesh(**_MESH),
        compiler_params=_SC_PARAMS,
        scratch_types=[
            pltpu.VMEM((VP,), jnp.float32),
            pltpu.VMEM((NL,), jnp.int32),
            pltpu.VMEM((ECH,), jnp.int32),
            pltpu.VMEM((ECH,), jnp.int32),
            pltpu.VMEM((ECH,), jnp.float32),
            pltpu.VMEM((ECH,), jnp.float32),
        ],
    )
    def k(rows_i, co_i, wm_i, cnt_i, dinv_i, nrm_o, dv, cb, rb, cob, wb, nb):
        wid = _wid()
        base = wid * NVT
        pltpu.sync_copy(dinv_i, dv)
        pltpu.sync_copy(cnt_i.at[wid], cb)
        n = cb[pl.ds(0, NL)][0]

        def ch(ci, _):
            o = ci * ECH
            pltpu.sync_copy(rows_i.at[wid, pl.ds(o, ECH)], rb)
            pltpu.sync_copy(co_i.at[wid, pl.ds(o, ECH)], cob)
            pltpu.sync_copy(wm_i.at[wid, pl.ds(o, ECH)], wb)
            for s in range(ECH // NL):
                r16 = rb[pl.ds(s * NL, NL)]
                c16 = (cob[pl.ds(s * NL, NL)] >> 7) + base
                w16 = wb[pl.ds(s * NL, NL)]
                dr = plsc.load_gather(dv, [r16])
                dc = plsc.load_gather(dv, [c16])
                nb[pl.ds(s * NL, NL)] = -(dr * w16 * dc)
            pltpu.sync_copy(nb, nrm_o.at[wid, pl.ds(o, ECH)])
            return 0
        lax.fori_loop(0, n // ECH, ch, 0)

    return k(rows, co, wm, cnt, dinv)


def _sc_prop(h3, rows, co, nrm, cnt, VP, NVT):
    nfc = h3.shape[0]
    h2 = h3.reshape(nfc * VP, 128)

    @functools.partial(
        pl.kernel,
        out_type=jax.ShapeDtypeStruct((nfc * VP * 128,), jnp.float32),
        mesh=plsc.VectorSubcoreMesh(**_MESH),
        compiler_params=_SC_PARAMS,
        scratch_types=[
            pltpu.VMEM((NL,), jnp.int32),
            pltpu.VMEM((ECH,), jnp.int32),
            pltpu.VMEM((ECH,), jnp.int32),
            pltpu.VMEM((ECH,), jnp.int32),
            pltpu.VMEM((ECH,), jnp.int32),
            pltpu.VMEM((ECH,), jnp.float32),
            pltpu.VMEM((ECH,), jnp.float32),
            pltpu.VMEM((ECH, 128), jnp.float32),
            pltpu.VMEM((ECH, 128), jnp.float32),
            pltpu.VMEM((NVT * 128,), jnp.float32),
            pltpu.SemaphoreType.DMA,
            pltpu.SemaphoreType.DMA,
        ],
    )
    def k(h_i, rows_i, co_i, nrm_i, cnt_i, p_o,
          cb, riA, riB, coA, coB, nrA, nrB, gA, gB, outb, semA, semB):
        wid = _wid()
        base = wid * NVT
        pltpu.sync_copy(cnt_i.at[wid], cb)
        npairs = cb[pl.ds(0, NL)][0] // PADQ
        zf = jnp.zeros((NL,), jnp.float32)

        def do_half(cof, nrf, g):
            def sub(sc, _):
                ad16 = cof[pl.ds(sc * NL, NL)]
                nr16 = nrf[pl.ds(sc * NL, NL)]
                for lane in range(NL):
                    e = sc * NL + lane
                    ad = ad16[lane]
                    nv = jnp.full((NL,), 0.0) + nr16[lane]
                    for j in range(128 // NL):
                        gv = g[e, pl.ds(j * NL, NL)]
                        plsc.addupdate(outb.at[pl.ds(ad + j * NL, NL)],
                                       nv * gv)
                return 0
            lax.fori_loop(0, ECH // NL, sub, 0)

        def fcloop(fc, _):
            fcoff = jnp.full((NL,), 0, jnp.int32) + fc * VP

            def zo(i, _):
                outb[pl.ds(i * NL, NL)] = zf
                return 0
            lax.fori_loop(0, NVT * 128 // NL, zo, 0)

            def fetch_idx(o, ri):
                pltpu.sync_copy(rows_i.at[wid, pl.ds(o, ECH)], ri)
                for s in range(ECH // NL):
                    ri[pl.ds(s * NL, NL)] = ri[pl.ds(s * NL, NL)] + fcoff

            def pair(pi, _):
                oA = pi * PADQ
                oB = oA + ECH
                fetch_idx(oA, riA)
                cpA = pltpu.async_copy(h_i.at[riA], gA, semA)
                fetch_idx(oB, riB)
                cpB = pltpu.async_copy(h_i.at[riB], gB, semB)
                pltpu.sync_copy(co_i.at[wid, pl.ds(oA, ECH)], coA)
                pltpu.sync_copy(nrm_i.at[wid, pl.ds(oA, ECH)], nrA)
                pltpu.sync_copy(co_i.at[wid, pl.ds(oB, ECH)], coB)
                pltpu.sync_copy(nrm_i.at[wid, pl.ds(oB, ECH)], nrB)
                cpA.wait()
                do_half(coA, nrA, gA)
                cpB.wait()
                do_half(coB, nrB, gB)
                return 0
            lax.fori_loop(0, npairs, pair, 0)
            pltpu.sync_copy(
                outb, p_o.at[pl.ds((fc * VP + base) * 128, NVT * 128)])
            return 0
        lax.fori_loop(0, nfc, fcloop, 0)

    return k(h2, rows, co, nrm, cnt).reshape(nfc, VP, 128)


def _glu(acc):
    p = acc[:, :C]
    q = acc[:, C:2 * C]
    r = acc[:, 2 * C:]
    return jnp.maximum(p * jax.nn.sigmoid(q) + r, 0.0)


def _tck_tconv_x(xt, Wc, bc, VP):
    B, T, V, F = xt.shape
    To = T - KT + 1
    nfc = (B * To * C) // 128

    def body(x_ref, w_ref, b_ref, o_ref):
        xr = x_ref[0]
        w = w_ref[...]
        bb = b_ref[...]
        for t in range(To):
            a = jnp.concatenate([xr[t + k] for k in range(KT)], axis=1)
            acc = jnp.dot(a.astype(MM_DT), w,
                          preferred_element_type=jnp.float32) + bb
            o_ref[t // 2, :, (t % 2) * C:(t % 2) * C + C] = _glu(acc)

    return pl.pallas_call(
        body,
        grid=(B, V // VT),
        in_specs=[
            pl.BlockSpec((1, T, VT, F), lambda b, i: (b, 0, i, 0)),
            pl.BlockSpec((KT * F, 3 * C), lambda b, i: (0, 0)),
            pl.BlockSpec((1, 3 * C), lambda b, i: (0, 0)),
        ],
        out_specs=pl.BlockSpec((To // 2, VT, 128), lambda b, i: (b, i, 0)),
        out_shape=jax.ShapeDtypeStruct((nfc, VP, 128), jnp.float32),
        compiler_params=_TC_PARAMS,
    )(xt, Wc, bc.reshape(1, 3 * C))


def _tck_tconv_v(xin, Wc, bc, VP):
    V, B, Tin, _ = xin.shape
    To = Tin - KT + 1
    nfc = (B * To * C) // 128

    def body(x_ref, w_ref, b_ref, o_ref):
        xr = x_ref[:, 0]
        w = w_ref[...]
        bb = b_ref[...]
        for t in range(To):
            a = jnp.concatenate([xr[:, t + k, :] for k in range(KT)], axis=1)
            acc = jnp.dot(a.astype(MM_DT), w,
                          preferred_element_type=jnp.float32) + bb
            o_ref[t // 2, :, (t % 2) * C:(t % 2) * C + C] = _glu(acc)

    return pl.pallas_call(
        body,
        grid=(B, V // VT),
        in_specs=[
            pl.BlockSpec((VT, 1, Tin, C), lambda b, i: (i, b, 0, 0)),
            pl.BlockSpec((KT * C, 3 * C), lambda b, i: (0, 0)),
            pl.BlockSpec((1, 3 * C), lambda b, i: (0, 0)),
        ],
        out_specs=pl.BlockSpec((To // 2, VT, 128), lambda b, i: (b, i, 0)),
        out_shape=jax.ShapeDtypeStruct((nfc, VP, 128), jnp.float32),
        compiler_params=_TC_PARAMS,
    )(xin, Wc, bc.reshape(1, 3 * C))


def _tck_cheb(T0, P1, P2, Wk, bk, B, To, V):
    nfc = T0.shape[0]
    Wcat = jnp.concatenate(
        [(Wk[0] - Wk[2]).T, Wk[1].T, 2.0 * Wk[2].T], axis=0).astype(MM_DT)

    def body(t0_ref, p1_ref, p2_ref, w_ref, b_ref, o_ref):
        w = w_ref[...]
        bb = b_ref[...]
        for b in range(B):
            for t in range(To):
                f = b * To + t
                fc, h = f // 2, (f % 2) * C
                xc = jnp.concatenate(
                    [t0_ref[fc, :, h:h + C], p1_ref[fc, :, h:h + C],
                     p2_ref[fc, :, h:h + C]], axis=1)
                acc = jnp.dot(xc.astype(MM_DT), w,
                              preferred_element_type=jnp.float32) + bb
                o_ref[:, b, t, :] = jnp.maximum(acc, 0.0)

    spec3 = pl.BlockSpec((nfc, VT, 128), lambda i: (0, i, 0))
    return pl.pallas_call(
        body,
        grid=(V // VT,),
        in_specs=[
            spec3, spec3, spec3,
            pl.BlockSpec((KG * C, C), lambda i: (0, 0)),
            pl.BlockSpec((1, C), lambda i: (0, 0)),
        ],
        out_specs=pl.BlockSpec((VT, B, To, C), lambda i: (i, 0, 0, 0)),
        out_shape=jax.ShapeDtypeStruct((V, B, To, C), jnp.float32),
        compiler_params=_TC_PARAMS,
    )(T0, P1, P2, Wcat, bk.reshape(1, C))


def _tck_tconv_bn(tg, Wc, bc, g, bb):
    V, B, Tin, _ = tg.shape
    To = Tin - KT + 1

    def body(x_ref, w_ref, bc_ref, g_ref, bb_ref, o_ref):
        w = w_ref[...]
        bcv = bc_ref[...]
        hs = []
        for b in range(B):
            for t in range(To):
                a = jnp.concatenate(
                    [x_ref[:, b, t + k, :] for k in range(KT)], axis=1)
                acc = jnp.dot(a.astype(MM_DT), w,
                              preferred_element_type=jnp.float32) + bcv
                hs.append(_glu(acc))
        hall = jnp.concatenate(hs, axis=1)
        mu = jnp.mean(hall, axis=1, keepdims=True)
        var = jnp.mean((hall - mu) ** 2, axis=1, keepdims=True)
        scl = lax.rsqrt(var + 1e-5) * g_ref[...]
        sh = bb_ref[...]
        i = 0
        for b in range(B):
            for t in range(To):
                o_ref[:, b, t, :] = (hs[i] - mu) * scl + sh
                i += 1

    return pl.pallas_call(
        body,
        grid=(V // VT,),
        in_specs=[
            pl.BlockSpec((VT, B, Tin, C), lambda i: (i, 0, 0, 0)),
            pl.BlockSpec((KT * C, 3 * C), lambda i: (0, 0)),
            pl.BlockSpec((1, 3 * C), lambda i: (0, 0)),
            pl.BlockSpec((VT, 1), lambda i: (i, 0)),
            pl.BlockSpec((VT, 1), lambda i: (i, 0)),
        ],
        out_specs=pl.BlockSpec((VT, B, To, C), lambda i: (i, 0, 0, 0)),
        out_shape=jax.ShapeDtypeStruct((V, B, To, C), jnp.float32),
        compiler_params=_TC_PARAMS,
    )(tg, Wc, bc.reshape(1, 3 * C), g.reshape(V, 1), bb.reshape(V, 1))


def _tck_linear(h2, lw, lb):
    V, B, Tf, _ = h2.shape
    wl = lw.T.astype(MM_DT)
    no = lw.shape[0]

    def body(x_ref, w_ref, b_ref, o_ref):
        xr = x_ref[...].reshape(VT, B, Tf * C)
        w = w_ref[...]
        bb = b_ref[...]
        for b in range(B):
            o_ref[b] = jnp.dot(xr[:, b].astype(MM_DT), w,
                               preferred_element_type=jnp.float32) + bb

    return pl.pallas_call(
        body,
        grid=(V // VT,),
        in_specs=[
            pl.BlockSpec((VT, B, Tf, C), lambda i: (i, 0, 0, 0)),
            pl.BlockSpec((Tf * C, no), lambda i: (0, 0)),
            pl.BlockSpec((1, no), lambda i: (0, 0)),
        ],
        out_specs=pl.BlockSpec((B, VT, no), lambda i: (0, i, 0)),
        out_shape=jax.ShapeDtypeStruct((B, V, no), jnp.float32),
        compiler_params=_TC_PARAMS,
    )(h2, wl, lb.reshape(1, no))


def _wcat(p, s, stage, cin):
    ws = [p["sc%d_t%d_w%d" % (s, stage, j)] for j in (1, 2, 3)]
    wc = jnp.concatenate(
        [jnp.concatenate([w[:, :, 0, k].T for w in ws], axis=1)
         for k in range(KT)], axis=0)
    bc = jnp.concatenate([p["sc%d_t%d_b%d" % (s, stage, j)]
                          for j in (1, 2, 3)], axis=0)
    return wc.astype(MM_DT), bc


def kernel(x, edge_index, edge_weight, params):
    B, V, F, T = x.shape
    NVT = ((V + NW - 1) // NW + 7) // 8 * 8
    VP = NW * NVT
    p = params

    xt = jnp.transpose(x, (0, 3, 1, 2))
    rows, co, wm, cnt, deg = _sc_prep(edge_index[0], edge_index[1],
                                      edge_weight, VP, NVT)
    dinv = _tc_dinv(deg, VP)
    nrm = _sc_norm(rows, co, wm, cnt, dinv, VP, NVT)

    h = None
    for s, tin in ((1, T), (2, T - 4)):
        wc1, bc1 = _wcat(p, s, 1, F if s == 1 else C)
        if s == 1:
            t0 = _tck_tconv_x(xt, wc1, bc1, VP)
        else:
            t0 = _tck_tconv_v(h, wc1, bc1, VP)
        to = tin - KT + 1
        p1 = _sc_prop(t0, rows, co, nrm, cnt, VP, NVT)
        p2 = _sc_prop(p1, rows, co, nrm, cnt, VP, NVT)
        tg = _tck_cheb(t0, p1, p2, p["sc%d_cheb_w" % s],
                       p["sc%d_cheb_b" % s], B, to, V)
        wc2, bc2 = _wcat(p, s, 2, C)
        h = _tck_tconv_bn(tg, wc2, bc2, p["sc%d_bn_g" % s],
                          p["sc%d_bn_b" % s])

    return _tck_linear(h, params["lin_w"], params["lin_b"])


# speedup vs baseline: 3.4383x; 1.0442x over previous
import functools

import jax
import jax.numpy as jnp
from jax import lax
from jax.experimental import pallas as pl
from jax.experimental.pallas import tpu as pltpu
from jax.experimental.pallas import tpu_sc as plsc

MM_DT = jnp.float32

KT = 3
KG = 3
C = 64

NC, NS, NL = 2, 16, 16
NW = NC * NS
ECAP = 16384
ECH = 128
PADQ = 2 * ECH
EB = 2000
VT = 400

_MESH = dict(core_axis_name="c", subcore_axis_name="s")
_SC_PARAMS = pltpu.CompilerParams(needs_layout_passes=False)
_TC_PARAMS = pltpu.CompilerParams(vmem_limit_bytes=100 * 1024 * 1024)


def _wid():
    return lax.axis_index("s") * NC + lax.axis_index("c")


def _sc_prep(erow, ecol, edge_weight, VP, NVT):
    E = edge_weight.shape[0]
    nblk = E // EB
    assert E % EB == 0 and E % NL == 0

    @functools.partial(
        pl.kernel,
        out_type=(
            jax.ShapeDtypeStruct((NW, ECAP), jnp.int32),
            jax.ShapeDtypeStruct((NW, ECAP), jnp.int32),
            jax.ShapeDtypeStruct((NW, ECAP), jnp.float32),
            jax.ShapeDtypeStruct((NW, NL), jnp.int32),
            jax.ShapeDtypeStruct((VP,), jnp.float32),
        ),
        mesh=plsc.VectorSubcoreMesh(**_MESH),
        compiler_params=_SC_PARAMS,
        scratch_types=[
            pltpu.VMEM((EB,), jnp.int32),
            pltpu.VMEM((EB,), jnp.int32),
            pltpu.VMEM((EB,), jnp.float32),
            pltpu.VMEM((ECAP,), jnp.int32),
            pltpu.VMEM((ECAP,), jnp.int32),
            pltpu.VMEM((ECAP,), jnp.float32),
            pltpu.VMEM((NVT * NL,), jnp.float32),
            pltpu.VMEM((NVT,), jnp.float32),
            pltpu.VMEM((NL,), jnp.int32),
        ],
    )
    def k(er, ec, ew, rows_o, co_o, wm_o, cnt_o, deg_o,
          rbuf, cbuf, wbuf, lrow, lco, lwm, deg16, degv, c16):
        wid = _wid()
        base = wid * NVT
        zi = jnp.zeros((NL,), jnp.int32)
        zf = jnp.zeros((NL,), jnp.float32)

        def zlist(i, _):
            lrow[pl.ds(i * NL, NL)] = zi
            lco[pl.ds(i * NL, NL)] = zi
            lwm[pl.ds(i * NL, NL)] = zf
            return 0
        lax.fori_loop(0, ECAP // NL, zlist, 0)

        def zdeg(i, _):
            deg16[pl.ds(i * NL, NL)] = zf
            return 0
        lax.fori_loop(0, NVT, zdeg, 0)

        lane_off = lax.iota(jnp.int32, NL) * NVT

        def blk(bi, off):
            pltpu.sync_copy(er.at[pl.ds(bi * EB, EB)], rbuf)
            pltpu.sync_copy(ec.at[pl.ds(bi * EB, EB)], cbuf)
            pltpu.sync_copy(ew.at[pl.ds(bi * EB, EB)], wbuf)

            def ch(ki, off):
                r = rbuf[pl.ds(ki * NL, NL)]
                c = cbuf[pl.ds(ki * NL, NL)]
                w = wbuf[pl.ds(ki * NL, NL)]
                nonself = r != c
                crel = c - base
                mc = (crel >= 0) & (crel < NVT)
                cum = plsc.cumsum(mc.astype(jnp.int32))
                pos = off + cum - 1
                plsc.store_scatter(lrow, [pos], r, mask=mc)
                plsc.store_scatter(lco, [pos], crel * 128, mask=mc)
                plsc.store_scatter(lwm, [pos], jnp.where(nonself, w, 0.0),
                                   mask=mc)
                rrel = r - base
                mr = (rrel >= 0) & (rrel < NVT) & nonself
                idx = jnp.where(mr, rrel, 0) + lane_off
                plsc.addupdate_scatter(deg16, [idx], w, mask=mr)
                return off + cum[NL - 1]
            return lax.fori_loop(0, EB // NL, ch, off)

        off = lax.fori_loop(0, nblk, blk, jnp.int32(0))

        for vc in range(NVT // NL):
            acc = zf
            for cp in range(NL):
                acc = acc + deg16[pl.ds(cp * NVT + vc * NL, NL)]
            degv[pl.ds(vc * NL, NL)] = acc

        poff = ((off + PADQ - 1) // PADQ) * PADQ
        c16[...] = jnp.full((NL,), 0, jnp.int32) + poff
        pltpu.sync_copy(c16, cnt_o.at[wid])
        pltpu.sync_copy(lrow, rows_o.at[wid])
        pltpu.sync_copy(lco, co_o.at[wid])
        pltpu.sync_copy(lwm, wm_o.at[wid])
        pltpu.sync_copy(degv, deg_o.at[pl.ds(base, NVT)])

    return k(erow, ecol, edge_weight)


def _tc_dinv(deg, VP):
    def body(d_ref, o_ref):
        d = d_ref[...]
        o_ref[...] = jnp.where(d > 0, lax.rsqrt(jnp.where(d > 0, d, 1.0)),
                               0.0)
    out = pl.pallas_call(
        body,
        out_shape=jax.ShapeDtypeStruct((VP // 128, 128), jnp.float32),
    )(deg.reshape(VP // 128, 128))
    return out.reshape(VP)


def _sc_norm(rows, co, wm, cnt, dinv, VP, NVT):
    @functools.partial(
        pl.kernel,
        out_type=jax.ShapeDtypeStruct((NW, ECAP), jnp.float32),
        mesh=plsc.VectorSubcoreMesh(**_MESH),
        compiler_params=_SC_PARAMS,
        scratch_types=[
            pltpu.VMEM((VP,), jnp.float32),
            pltpu.VMEM((NL,), jnp.int32),
            pltpu.VMEM((ECH,), jnp.int32),
            pltpu.VMEM((ECH,), jnp.int32),
            pltpu.VMEM((ECH,), jnp.float32),
            pltpu.VMEM((ECH,), jnp.float32),
        ],
    )
    def k(rows_i, co_i, wm_i, cnt_i, dinv_i, nrm_o, dv, cb, rb, cob, wb, nb):
        wid = _wid()
        base = wid * NVT
        pltpu.sync_copy(dinv_i, dv)
        pltpu.sync_copy(cnt_i.at[wid], cb)
        n = cb[pl.ds(0, NL)][0]

        def ch(ci, _):
            o = ci * ECH
            pltpu.sync_copy(rows_i.at[wid, pl.ds(o, ECH)], rb)
            pltpu.sync_copy(co_i.at[wid, pl.ds(o, ECH)], cob)
            pltpu.sync_copy(wm_i.at[wid, pl.ds(o, ECH)], wb)
            for s in range(ECH // NL):
                r16 = rb[pl.ds(s * NL, NL)]
                c16 = (cob[pl.ds(s * NL, NL)] >> 7) + base
                w16 = wb[pl.ds(s * NL, NL)]
                dr = plsc.load_gather(dv, [r16])
                dc = plsc.load_gather(dv, [c16])
                nb[pl.ds(s * NL, NL)] = -(dr * w16 * dc)
            pltpu.sync_copy(nb, nrm_o.at[wid, pl.ds(o, ECH)])
            return 0
        lax.fori_loop(0, n // ECH, ch, 0)

    return k(rows, co, wm, cnt, dinv)


def _sc_prop(h3, rows, co, nrm, cnt, VP, NVT):
    nfc = h3.shape[0]
    h2 = h3.reshape(nfc * VP, 128)

    @functools.partial(
        pl.kernel,
        out_type=jax.ShapeDtypeStruct((nfc * VP * 128,), jnp.float32),
        mesh=plsc.VectorSubcoreMesh(**_MESH),
        compiler_params=_SC_PARAMS,
        scratch_types=[
            pltpu.VMEM((NL,), jnp.int32),
            pltpu.VMEM((ECAP,), jnp.int32),
            pltpu.VMEM((ECAP,), jnp.int32),
            pltpu.VMEM((ECAP,), jnp.float32),
            pltpu.VMEM((ECH, 128), jnp.float32),
            pltpu.VMEM((ECH, 128), jnp.float32),
            pltpu.VMEM((NVT * 128,), jnp.float32),
            pltpu.SemaphoreType.DMA,
            pltpu.SemaphoreType.DMA,
        ],
    )
    def k(h_i, rows_i, co_i, nrm_i, cnt_i, p_o,
          cb, rbig, cbig, nbig, gA, gB, outb, semA, semB):
        wid = _wid()
        base = wid * NVT
        pltpu.sync_copy(cnt_i.at[wid], cb)
        n = cb[pl.ds(0, NL)][0]
        npairs = n // PADQ
        zf = jnp.zeros((NL,), jnp.float32)
        vpv = jnp.full((NL,), 0, jnp.int32) + VP
        pltpu.sync_copy(rows_i.at[wid], rbig)
        pltpu.sync_copy(co_i.at[wid], cbig)
        pltpu.sync_copy(nrm_i.at[wid], nbig)

        def do_half(obase, g):
            def sub(sc, _):
                o = obase + sc * NL
                ad16 = cbig[pl.ds(o, NL)]
                nr16 = nbig[pl.ds(o, NL)]
                for lane in range(NL):
                    e = sc * NL + lane
                    ad = ad16[lane]
                    nv = jnp.full((NL,), 0.0) + nr16[lane]
                    for j in range(128 // NL):
                        gv = g[e, pl.ds(j * NL, NL)]
                        plsc.addupdate(outb.at[pl.ds(ad + j * NL, NL)],
                                       nv * gv)
                return 0
            lax.fori_loop(0, ECH // NL, sub, 0)

        def fcloop(fc, _):
            def zo(i, _):
                outb[pl.ds(i * NL, NL)] = zf
                return 0
            lax.fori_loop(0, NVT * 128 // NL, zo, 0)

            def pair(pi, _):
                oA = pi * PADQ
                oB = oA + ECH
                cpA = pltpu.async_copy(h_i.at[rbig.at[pl.ds(oA, ECH)]],
                                       gA, semA)
                cpB = pltpu.async_copy(h_i.at[rbig.at[pl.ds(oB, ECH)]],
                                       gB, semB)
                cpA.wait()
                do_half(oA, gA)
                cpB.wait()
                do_half(oB, gB)
                return 0
            lax.fori_loop(0, npairs, pair, 0)
            pltpu.sync_copy(
                outb, p_o.at[pl.ds((fc * VP + base) * 128, NVT * 128)])

            def adv(i, _):
                rbig[pl.ds(i * NL, NL)] = rbig[pl.ds(i * NL, NL)] + vpv
                return 0
            lax.fori_loop(0, n // NL, adv, 0)
            return 0
        lax.fori_loop(0, nfc, fcloop, 0)

    return k(h2, rows, co, nrm, cnt).reshape(nfc, VP, 128)


def _glu(acc):
    p = acc[:, :C]
    q = acc[:, C:2 * C]
    r = acc[:, 2 * C:]
    return jnp.maximum(p * jax.nn.sigmoid(q) + r, 0.0)


def _tck_tconv_x(xt, Wc, bc, VP):
    B, T, V, F = xt.shape
    To = T - KT + 1
    nfc = (B * To * C) // 128

    def body(x_ref, w_ref, b_ref, o_ref):
        xr = x_ref[0]
        w = w_ref[...]
        bb = b_ref[...]
        for t in range(To):
            a = jnp.concatenate([xr[t + k] for k in range(KT)], axis=1)
            acc = jnp.dot(a.astype(MM_DT), w,
                          preferred_element_type=jnp.float32) + bb
            o_ref[t // 2, :, (t % 2) * C:(t % 2) * C + C] = _glu(acc)

    return pl.pallas_call(
        body,
        grid=(B, V // VT),
        in_specs=[
            pl.BlockSpec((1, T, VT, F), lambda b, i: (b, 0, i, 0)),
            pl.BlockSpec((KT * F, 3 * C), lambda b, i: (0, 0)),
            pl.BlockSpec((1, 3 * C), lambda b, i: (0, 0)),
        ],
        out_specs=pl.BlockSpec((To // 2, VT, 128), lambda b, i: (b, i, 0)),
        out_shape=jax.ShapeDtypeStruct((nfc, VP, 128), jnp.float32),
        compiler_params=_TC_PARAMS,
    )(xt, Wc, bc.reshape(1, 3 * C))


def _tck_tconv_v(xin, Wc, bc, VP):
    V, B, Tin, _ = xin.shape
    To = Tin - KT + 1
    nfc = (B * To * C) // 128

    def body(x_ref, w_ref, b_ref, o_ref):
        xr = x_ref[:, 0]
        w = w_ref[...]
        bb = b_ref[...]
        for t in range(To):
            a = jnp.concatenate([xr[:, t + k, :] for k in range(KT)], axis=1)
            acc = jnp.dot(a.astype(MM_DT), w,
                          preferred_element_type=jnp.float32) + bb
            o_ref[t // 2, :, (t % 2) * C:(t % 2) * C + C] = _glu(acc)

    return pl.pallas_call(
        body,
        grid=(B, V // VT),
        in_specs=[
            pl.BlockSpec((VT, 1, Tin, C), lambda b, i: (i, b, 0, 0)),
            pl.BlockSpec((KT * C, 3 * C), lambda b, i: (0, 0)),
            pl.BlockSpec((1, 3 * C), lambda b, i: (0, 0)),
        ],
        out_specs=pl.BlockSpec((To // 2, VT, 128), lambda b, i: (b, i, 0)),
        out_shape=jax.ShapeDtypeStruct((nfc, VP, 128), jnp.float32),
        compiler_params=_TC_PARAMS,
    )(xin, Wc, bc.reshape(1, 3 * C))


def _tck_cheb(T0, P1, P2, Wk, bk, B, To, V):
    nfc = T0.shape[0]
    Wcat = jnp.concatenate(
        [(Wk[0] - Wk[2]).T, Wk[1].T, 2.0 * Wk[2].T], axis=0).astype(MM_DT)

    def body(t0_ref, p1_ref, p2_ref, w_ref, b_ref, o_ref):
        w = w_ref[...]
        bb = b_ref[...]
        for b in range(B):
            for t in range(To):
                f = b * To + t
                fc, h = f // 2, (f % 2) * C
                xc = jnp.concatenate(
                    [t0_ref[fc, :, h:h + C], p1_ref[fc, :, h:h + C],
                     p2_ref[fc, :, h:h + C]], axis=1)
                acc = jnp.dot(xc.astype(MM_DT), w,
                              preferred_element_type=jnp.float32) + bb
                o_ref[:, b, t, :] = jnp.maximum(acc, 0.0)

    spec3 = pl.BlockSpec((nfc, VT, 128), lambda i: (0, i, 0))
    return pl.pallas_call(
        body,
        grid=(V // VT,),
        in_specs=[
            spec3, spec3, spec3,
            pl.BlockSpec((KG * C, C), lambda i: (0, 0)),
            pl.BlockSpec((1, C), lambda i: (0, 0)),
        ],
        out_specs=pl.BlockSpec((VT, B, To, C), lambda i: (i, 0, 0, 0)),
        out_shape=jax.ShapeDtypeStruct((V, B, To, C), jnp.float32),
        compiler_params=_TC_PARAMS,
    )(T0, P1, P2, Wcat, bk.reshape(1, C))


def _tck_tconv_bn(tg, Wc, bc, g, bb):
    V, B, Tin, _ = tg.shape
    To = Tin - KT + 1

    def body(x_ref, w_ref, bc_ref, g_ref, bb_ref, o_ref):
        w = w_ref[...]
        bcv = bc_ref[...]
        hs = []
        for b in range(B):
            for t in range(To):
                a = jnp.concatenate(
                    [x_ref[:, b, t + k, :] for k in range(KT)], axis=1)
                acc = jnp.dot(a.astype(MM_DT), w,
                              preferred_element_type=jnp.float32) + bcv
                hs.append(_glu(acc))
        hall = jnp.concatenate(hs, axis=1)
        mu = jnp.mean(hall, axis=1, keepdims=True)
        var = jnp.mean((hall - mu) ** 2, axis=1, keepdims=True)
        scl = lax.rsqrt(var + 1e-5) * g_ref[...]
        sh = bb_ref[...]
        i = 0
        for b in range(B):
            for t in range(To):
                o_ref[:, b, t, :] = (hs[i] - mu) * scl + sh
                i += 1

    return pl.pallas_call(
        body,
        grid=(V // VT,),
        in_specs=[
            pl.BlockSpec((VT, B, Tin, C), lambda i: (i, 0, 0, 0)),
            pl.BlockSpec((KT * C, 3 * C), lambda i: (0, 0)),
            pl.BlockSpec((1, 3 * C), lambda i: (0, 0)),
            pl.BlockSpec((VT, 1), lambda i: (i, 0)),
            pl.BlockSpec((VT, 1), lambda i: (i, 0)),
        ],
        out_specs=pl.BlockSpec((VT, B, To, C), lambda i: (i, 0, 0, 0)),
        out_shape=jax.ShapeDtypeStruct((V, B, To, C), jnp.float32),
        compiler_params=_TC_PARAMS,
    )(tg, Wc, bc.reshape(1, 3 * C), g.reshape(V, 1), bb.reshape(V, 1))


def _tck_linear(h2, lw, lb):
    V, B, Tf, _ = h2.shape
    wl = lw.T.astype(MM_DT)
    no = lw.shape[0]

    def body(x_ref, w_ref, b_ref, o_ref):
        xr = x_ref[...].reshape(VT, B, Tf * C)
        w = w_ref[...]
        bb = b_ref[...]
        for b in range(B):
            o_ref[b] = jnp.dot(xr[:, b].astype(MM_DT), w,
                               preferred_element_type=jnp.float32) + bb

    return pl.pallas_call(
        body,
        grid=(V // VT,),
        in_specs=[
            pl.BlockSpec((VT, B, Tf, C), lambda i: (i, 0, 0, 0)),
            pl.BlockSpec((Tf * C, no), lambda i: (0, 0)),
            pl.BlockSpec((1, no), lambda i: (0, 0)),
        ],
        out_specs=pl.BlockSpec((B, VT, no), lambda i: (0, i, 0)),
        out_shape=jax.ShapeDtypeStruct((B, V, no), jnp.float32),
        compiler_params=_TC_PARAMS,
    )(h2, wl, lb.reshape(1, no))


def _wcat(p, s, stage, cin):
    ws = [p["sc%d_t%d_w%d" % (s, stage, j)] for j in (1, 2, 3)]
    wc = jnp.concatenate(
        [jnp.concatenate([w[:, :, 0, k].T for w in ws], axis=1)
         for k in range(KT)], axis=0)
    bc = jnp.concatenate([p["sc%d_t%d_b%d" % (s, stage, j)]
                          for j in (1, 2, 3)], axis=0)
    return wc.astype(MM_DT), bc


def kernel(x, edge_index, edge_weight, params):
    B, V, F, T = x.shape
    NVT = ((V + NW - 1) // NW + 7) // 8 * 8
    VP = NW * NVT
    p = params

    xt = jnp.transpose(x, (0, 3, 1, 2))
    rows, co, wm, cnt, deg = _sc_prep(edge_index[0], edge_index[1],
                                      edge_weight, VP, NVT)
    dinv = _tc_dinv(deg, VP)
    nrm = _sc_norm(rows, co, wm, cnt, dinv, VP, NVT)

    h = None
    for s, tin in ((1, T), (2, T - 4)):
        wc1, bc1 = _wcat(p, s, 1, F if s == 1 else C)
        if s == 1:
            t0 = _tck_tconv_x(xt, wc1, bc1, VP)
        else:
            t0 = _tck_tconv_v(h, wc1, bc1, VP)
        to = tin - KT + 1
        p1 = _sc_prop(t0, rows, co, nrm, cnt, VP, NVT)
        p2 = _sc_prop(p1, rows, co, nrm, cnt, VP, NVT)
        tg = _tck_cheb(t0, p1, p2, p["sc%d_cheb_w" % s],
                       p["sc%d_cheb_b" % s], B, to, V)
        wc2, bc2 = _wcat(p, s, 2, C)
        h = _tck_tconv_bn(tg, wc2, bc2, p["sc%d_bn_g" % s],
                          p["sc%d_bn_b" % s])

    return _tck_linear(h, params["lin_w"], params["lin_b"])


# speedup vs baseline: 5.5997x; 1.6286x over previous
import functools

import jax
import jax.numpy as jnp
from jax import lax
from jax.experimental import pallas as pl
from jax.experimental.pallas import tpu as pltpu
from jax.experimental.pallas import tpu_sc as plsc

MM_DT = jnp.float32

KT = 3
KG = 3
C = 64

NC, NS, NL = 2, 16, 16
NW = NC * NS
ECAP = 16384
ECH = 128
PADQ = 2 * ECH
EB = 2000
VT = 400

_MESH = dict(core_axis_name="c", subcore_axis_name="s")
_SC_PARAMS = pltpu.CompilerParams(needs_layout_passes=False)
_TC_PARAMS = pltpu.CompilerParams(vmem_limit_bytes=100 * 1024 * 1024)


def _wid():
    return lax.axis_index("s") * NC + lax.axis_index("c")


def _sc_prep(erow, ecol, edge_weight, VP, NVT):
    E = edge_weight.shape[0]
    nblk = E // EB
    assert E % EB == 0 and E % NL == 0

    @functools.partial(
        pl.kernel,
        out_type=(
            jax.ShapeDtypeStruct((NW, ECAP), jnp.int32),
            jax.ShapeDtypeStruct((NW, ECAP), jnp.int32),
            jax.ShapeDtypeStruct((NW, ECAP), jnp.float32),
            jax.ShapeDtypeStruct((NW, NL), jnp.int32),
            jax.ShapeDtypeStruct((VP,), jnp.float32),
        ),
        mesh=plsc.VectorSubcoreMesh(**_MESH),
        compiler_params=_SC_PARAMS,
        scratch_types=[
            pltpu.VMEM((EB,), jnp.int32),
            pltpu.VMEM((EB,), jnp.int32),
            pltpu.VMEM((EB,), jnp.float32),
            pltpu.VMEM((ECAP,), jnp.int32),
            pltpu.VMEM((ECAP,), jnp.int32),
            pltpu.VMEM((ECAP,), jnp.float32),
            pltpu.VMEM((NVT * NL,), jnp.float32),
            pltpu.VMEM((NVT,), jnp.float32),
            pltpu.VMEM((NL,), jnp.int32),
        ],
    )
    def k(er, ec, ew, rows_o, co_o, wm_o, cnt_o, deg_o,
          rbuf, cbuf, wbuf, lrow, lco, lwm, deg16, degv, c16):
        wid = _wid()
        base = wid * NVT
        zi = jnp.zeros((NL,), jnp.int32)
        zf = jnp.zeros((NL,), jnp.float32)

        def zlist(i, _):
            lrow[pl.ds(i * NL, NL)] = zi
            lco[pl.ds(i * NL, NL)] = zi
            lwm[pl.ds(i * NL, NL)] = zf
            return 0
        lax.fori_loop(0, ECAP // NL, zlist, 0)

        def zdeg(i, _):
            deg16[pl.ds(i * NL, NL)] = zf
            return 0
        lax.fori_loop(0, NVT, zdeg, 0)

        lane_off = lax.iota(jnp.int32, NL) * NVT

        def blk(bi, off):
            pltpu.sync_copy(er.at[pl.ds(bi * EB, EB)], rbuf)
            pltpu.sync_copy(ec.at[pl.ds(bi * EB, EB)], cbuf)
            pltpu.sync_copy(ew.at[pl.ds(bi * EB, EB)], wbuf)

            def ch(ki, off):
                r = rbuf[pl.ds(ki * NL, NL)]
                c = cbuf[pl.ds(ki * NL, NL)]
                w = wbuf[pl.ds(ki * NL, NL)]
                nonself = r != c
                crel = c - base
                mc = (crel >= 0) & (crel < NVT)
                cum = plsc.cumsum(mc.astype(jnp.int32))
                pos = off + cum - 1
                plsc.store_scatter(lrow, [pos], r, mask=mc)
                plsc.store_scatter(lco, [pos], crel * 128, mask=mc)
                plsc.store_scatter(lwm, [pos], jnp.where(nonself, w, 0.0),
                                   mask=mc)
                rrel = r - base
                mr = (rrel >= 0) & (rrel < NVT) & nonself
                idx = jnp.where(mr, rrel, 0) + lane_off
                plsc.addupdate_scatter(deg16, [idx], w, mask=mr)
                return off + cum[NL - 1]
            return lax.fori_loop(0, EB // NL, ch, off)

        off = lax.fori_loop(0, nblk, blk, jnp.int32(0))

        for vc in range(NVT // NL):
            acc = zf
            for cp in range(NL):
                acc = acc + deg16[pl.ds(cp * NVT + vc * NL, NL)]
            degv[pl.ds(vc * NL, NL)] = acc

        poff = ((off + PADQ - 1) // PADQ) * PADQ
        c16[...] = jnp.full((NL,), 0, jnp.int32) + poff
        pltpu.sync_copy(c16, cnt_o.at[wid])
        pltpu.sync_copy(lrow, rows_o.at[wid])
        pltpu.sync_copy(lco, co_o.at[wid])
        pltpu.sync_copy(lwm, wm_o.at[wid])
        pltpu.sync_copy(degv, deg_o.at[pl.ds(base, NVT)])

    return k(erow, ecol, edge_weight)


def _tc_dinv(deg, VP):
    def body(d_ref, o_ref):
        d = d_ref[...]
        o_ref[...] = jnp.where(d > 0, lax.rsqrt(jnp.where(d > 0, d, 1.0)),
                               0.0)
    out = pl.pallas_call(
        body,
        out_shape=jax.ShapeDtypeStruct((VP // 128, 128), jnp.float32),
    )(deg.reshape(VP // 128, 128))
    return out.reshape(VP)


def _sc_norm(rows, co, wm, cnt, dinv, VP, NVT):
    @functools.partial(
        pl.kernel,
        out_type=jax.ShapeDtypeStruct((NW, ECAP), jnp.float32),
        mesh=plsc.VectorSubcoreMesh(**_MESH),
        compiler_params=_SC_PARAMS,
        scratch_types=[
            pltpu.VMEM((VP,), jnp.float32),
            pltpu.VMEM((NL,), jnp.int32),
            pltpu.VMEM((ECH,), jnp.int32),
            pltpu.VMEM((ECH,), jnp.int32),
            pltpu.VMEM((ECH,), jnp.float32),
            pltpu.VMEM((ECH,), jnp.float32),
        ],
    )
    def k(rows_i, co_i, wm_i, cnt_i, dinv_i, nrm_o, dv, cb, rb, cob, wb, nb):
        wid = _wid()
        base = wid * NVT
        pltpu.sync_copy(dinv_i, dv)
        pltpu.sync_copy(cnt_i.at[wid], cb)
        n = cb[pl.ds(0, NL)][0]

        def ch(ci, _):
            o = ci * ECH
            pltpu.sync_copy(rows_i.at[wid, pl.ds(o, ECH)], rb)
            pltpu.sync_copy(co_i.at[wid, pl.ds(o, ECH)], cob)
            pltpu.sync_copy(wm_i.at[wid, pl.ds(o, ECH)], wb)
            for s in range(ECH // NL):
                r16 = rb[pl.ds(s * NL, NL)]
                c16 = (cob[pl.ds(s * NL, NL)] >> 7) + base
                w16 = wb[pl.ds(s * NL, NL)]
                dr = plsc.load_gather(dv, [r16])
                dc = plsc.load_gather(dv, [c16])
                nb[pl.ds(s * NL, NL)] = -(dr * w16 * dc)
            pltpu.sync_copy(nb, nrm_o.at[wid, pl.ds(o, ECH)])
            return 0
        lax.fori_loop(0, n // ECH, ch, 0)

    return k(rows, co, wm, cnt, dinv)


def _sc_prop(h3, rows, co, nrm, cnt, VP, NVT):
    nfc = h3.shape[0]
    h2 = h3.reshape(nfc * VP, 128)

    @functools.partial(
        pl.kernel,
        out_type=jax.ShapeDtypeStruct((nfc * VP * 128,), jnp.float32),
        mesh=plsc.VectorSubcoreMesh(**_MESH),
        compiler_params=_SC_PARAMS,
        scratch_types=[
            pltpu.VMEM((NL,), jnp.int32),
            pltpu.VMEM((ECAP,), jnp.int32),
            pltpu.VMEM((ECAP,), jnp.int32),
            pltpu.VMEM((ECAP,), jnp.float32),
            pltpu.VMEM((ECH, 128), jnp.float32),
            pltpu.VMEM((ECH, 128), jnp.float32),
            pltpu.VMEM((NVT * 128,), jnp.float32),
            pltpu.SemaphoreType.DMA,
            pltpu.SemaphoreType.DMA,
        ],
    )
    def k(h_i, rows_i, co_i, nrm_i, cnt_i, p_o,
          cb, rbig, cbig, nbig, gA, gB, outb, semA, semB):
        wid = _wid()
        base = wid * NVT
        pltpu.sync_copy(cnt_i.at[wid], cb)
        n = cb[pl.ds(0, NL)][0]
        npairs = n // PADQ
        zf = jnp.zeros((NL,), jnp.float32)
        vpv = jnp.full((NL,), 0, jnp.int32) + VP
        pltpu.sync_copy(rows_i.at[wid], rbig)
        pltpu.sync_copy(co_i.at[wid], cbig)
        pltpu.sync_copy(nrm_i.at[wid], nbig)

        def do_half(obase, g):
            def sub(sc, _):
                o = obase + sc * NL
                ad16 = cbig[pl.ds(o, NL)]
                nr16 = nbig[pl.ds(o, NL)]
                for lane in range(NL):
                    e = sc * NL + lane
                    ad = ad16[lane]
                    nv = jnp.full((NL,), 0.0) + nr16[lane]
                    gvs = [g[e, pl.ds(j * NL, NL)]
                           for j in range(128 // NL)]
                    pvs = [nv * gv for gv in gvs]
                    for j in range(128 // NL):
                        plsc.addupdate(outb.at[pl.ds(ad + j * NL, NL)],
                                       pvs[j])
                return 0
            lax.fori_loop(0, ECH // NL, sub, 0)

        def fcloop(fc, _):
            def zo(i, _):
                outb[pl.ds(i * NL, NL)] = zf
                return 0
            lax.fori_loop(0, NVT * 128 // NL, zo, 0)

            def pair(pi, _):
                oA = pi * PADQ
                oB = oA + ECH
                cpA = pltpu.async_copy(h_i.at[rbig.at[pl.ds(oA, ECH)]],
                                       gA, semA)
                cpB = pltpu.async_copy(h_i.at[rbig.at[pl.ds(oB, ECH)]],
                                       gB, semB)
                cpA.wait()
                do_half(oA, gA)
                cpB.wait()
                do_half(oB, gB)
                return 0
            lax.fori_loop(0, npairs, pair, 0)
            pltpu.sync_copy(
                outb, p_o.at[pl.ds((fc * VP + base) * 128, NVT * 128)])

            def adv(i, _):
                rbig[pl.ds(i * NL, NL)] = rbig[pl.ds(i * NL, NL)] + vpv
                return 0
            lax.fori_loop(0, n // NL, adv, 0)
            return 0
        lax.fori_loop(0, nfc, fcloop, 0)

    return k(h2, rows, co, nrm, cnt).reshape(nfc, VP, 128)


def _glu(acc):
    p = acc[:, :C]
    q = acc[:, C:2 * C]
    r = acc[:, 2 * C:]
    return jnp.maximum(p * jax.nn.sigmoid(q) + r, 0.0)


def _tck_tconv_x(xt, Wc, bc, VP):
    B, T, V, F = xt.shape
    To = T - KT + 1
    nfc = (B * To * C) // 128

    def body(x_ref, w_ref, b_ref, o_ref):
        xr = x_ref[0]
        w = w_ref[...]
        bb = b_ref[...]
        for t in range(To):
            a = jnp.concatenate([xr[t + k] for k in range(KT)], axis=1)
            acc = jnp.dot(a.astype(MM_DT), w,
                          preferred_element_type=jnp.float32) + bb
            o_ref[t // 2, :, (t % 2) * C:(t % 2) * C + C] = _glu(acc)

    return pl.pallas_call(
        body,
        grid=(B, V // VT),
        in_specs=[
            pl.BlockSpec((1, T, VT, F), lambda b, i: (b, 0, i, 0)),
            pl.BlockSpec((KT * F, 3 * C), lambda b, i: (0, 0)),
            pl.BlockSpec((1, 3 * C), lambda b, i: (0, 0)),
        ],
        out_specs=pl.BlockSpec((To // 2, VT, 128), lambda b, i: (b, i, 0)),
        out_shape=jax.ShapeDtypeStruct((nfc, VP, 128), jnp.float32),
        compiler_params=_TC_PARAMS,
    )(xt, Wc, bc.reshape(1, 3 * C))


def _tck_tconv_v(xin, Wc, bc, VP):
    V, B, Tin, _ = xin.shape
    To = Tin - KT + 1
    nfc = (B * To * C) // 128

    def body(x_ref, w_ref, b_ref, o_ref):
        xr = x_ref[:, 0]
        w = w_ref[...]
        bb = b_ref[...]
        for t in range(To):
            a = jnp.concatenate([xr[:, t + k, :] for k in range(KT)], axis=1)
            acc = jnp.dot(a.astype(MM_DT), w,
                          preferred_element_type=jnp.float32) + bb
            o_ref[t // 2, :, (t % 2) * C:(t % 2) * C + C] = _glu(acc)

    return pl.pallas_call(
        body,
        grid=(B, V // VT),
        in_specs=[
            pl.BlockSpec((VT, 1, Tin, C), lambda b, i: (i, b, 0, 0)),
            pl.BlockSpec((KT * C, 3 * C), lambda b, i: (0, 0)),
            pl.BlockSpec((1, 3 * C), lambda b, i: (0, 0)),
        ],
        out_specs=pl.BlockSpec((To // 2, VT, 128), lambda b, i: (b, i, 0)),
        out_shape=jax.ShapeDtypeStruct((nfc, VP, 128), jnp.float32),
        compiler_params=_TC_PARAMS,
    )(xin, Wc, bc.reshape(1, 3 * C))


def _tck_cheb(T0, P1, P2, Wk, bk, B, To, V):
    nfc = T0.shape[0]
    Wcat = jnp.concatenate(
        [(Wk[0] - Wk[2]).T, Wk[1].T, 2.0 * Wk[2].T], axis=0).astype(MM_DT)

    def body(t0_ref, p1_ref, p2_ref, w_ref, b_ref, o_ref):
        w = w_ref[...]
        bb = b_ref[...]
        for b in range(B):
            for t in range(To):
                f = b * To + t
                fc, h = f // 2, (f % 2) * C
                xc = jnp.concatenate(
                    [t0_ref[fc, :, h:h + C], p1_ref[fc, :, h:h + C],
                     p2_ref[fc, :, h:h + C]], axis=1)
                acc = jnp.dot(xc.astype(MM_DT), w,
                              preferred_element_type=jnp.float32) + bb
                o_ref[:, b, t, :] = jnp.maximum(acc, 0.0)

    spec3 = pl.BlockSpec((nfc, VT, 128), lambda i: (0, i, 0))
    return pl.pallas_call(
        body,
        grid=(V // VT,),
        in_specs=[
            spec3, spec3, spec3,
            pl.BlockSpec((KG * C, C), lambda i: (0, 0)),
            pl.BlockSpec((1, C), lambda i: (0, 0)),
        ],
        out_specs=pl.BlockSpec((VT, B, To, C), lambda i: (i, 0, 0, 0)),
        out_shape=jax.ShapeDtypeStruct((V, B, To, C), jnp.float32),
        compiler_params=_TC_PARAMS,
    )(T0, P1, P2, Wcat, bk.reshape(1, C))


def _tck_tconv_bn(tg, Wc, bc, g, bb):
    V, B, Tin, _ = tg.shape
    To = Tin - KT + 1

    def body(x_ref, w_ref, bc_ref, g_ref, bb_ref, o_ref):
        w = w_ref[...]
        bcv = bc_ref[...]
        hs = []
        for b in range(B):
            for t in range(To):
                a = jnp.concatenate(
                    [x_ref[:, b, t + k, :] for k in range(KT)], axis=1)
                acc = jnp.dot(a.astype(MM_DT), w,
                              preferred_element_type=jnp.float32) + bcv
                hs.append(_glu(acc))
        hall = jnp.concatenate(hs, axis=1)
        mu = jnp.mean(hall, axis=1, keepdims=True)
        var = jnp.mean((hall - mu) ** 2, axis=1, keepdims=True)
        scl = lax.rsqrt(var + 1e-5) * g_ref[...]
        sh = bb_ref[...]
        i = 0
        for b in range(B):
            for t in range(To):
                o_ref[:, b, t, :] = (hs[i] - mu) * scl + sh
                i += 1

    return pl.pallas_call(
        body,
        grid=(V // VT,),
        in_specs=[
            pl.BlockSpec((VT, B, Tin, C), lambda i: (i, 0, 0, 0)),
            pl.BlockSpec((KT * C, 3 * C), lambda i: (0, 0)),
            pl.BlockSpec((1, 3 * C), lambda i: (0, 0)),
            pl.BlockSpec((VT, 1), lambda i: (i, 0)),
            pl.BlockSpec((VT, 1), lambda i: (i, 0)),
        ],
        out_specs=pl.BlockSpec((VT, B, To, C), lambda i: (i, 0, 0, 0)),
        out_shape=jax.ShapeDtypeStruct((V, B, To, C), jnp.float32),
        compiler_params=_TC_PARAMS,
    )(tg, Wc, bc.reshape(1, 3 * C), g.reshape(V, 1), bb.reshape(V, 1))


def _tck_linear(h2, lw, lb):
    V, B, Tf, _ = h2.shape
    wl = lw.T.astype(MM_DT)
    no = lw.shape[0]

    def body(x_ref, w_ref, b_ref, o_ref):
        xr = x_ref[...].reshape(VT, B, Tf * C)
        w = w_ref[...]
        bb = b_ref[...]
        for b in range(B):
            o_ref[b] = jnp.dot(xr[:, b].astype(MM_DT), w,
                               preferred_element_type=jnp.float32) + bb

    return pl.pallas_call(
        body,
        grid=(V // VT,),
        in_specs=[
            pl.BlockSpec((VT, B, Tf, C), lambda i: (i, 0, 0, 0)),
            pl.BlockSpec((Tf * C, no), lambda i: (0, 0)),
            pl.BlockSpec((1, no), lambda i: (0, 0)),
        ],
        out_specs=pl.BlockSpec((B, VT, no), lambda i: (0, i, 0)),
        out_shape=jax.ShapeDtypeStruct((B, V, no), jnp.float32),
        compiler_params=_TC_PARAMS,
    )(h2, wl, lb.reshape(1, no))


def _wcat(p, s, stage, cin):
    ws = [p["sc%d_t%d_w%d" % (s, stage, j)] for j in (1, 2, 3)]
    wc = jnp.concatenate(
        [jnp.concatenate([w[:, :, 0, k].T for w in ws], axis=1)
         for k in range(KT)], axis=0)
    bc = jnp.concatenate([p["sc%d_t%d_b%d" % (s, stage, j)]
                          for j in (1, 2, 3)], axis=0)
    return wc.astype(MM_DT), bc


def kernel(x, edge_index, edge_weight, params):
    B, V, F, T = x.shape
    NVT = ((V + NW - 1) // NW + 7) // 8 * 8
    VP = NW * NVT
    p = params

    xt = jnp.transpose(x, (0, 3, 1, 2))
    rows, co, wm, cnt, deg = _sc_prep(edge_index[0], edge_index[1],
                                      edge_weight, VP, NVT)
    dinv = _tc_dinv(deg, VP)
    nrm = _sc_norm(rows, co, wm, cnt, dinv, VP, NVT)

    h = None
    for s, tin in ((1, T), (2, T - 4)):
        wc1, bc1 = _wcat(p, s, 1, F if s == 1 else C)
        if s == 1:
            t0 = _tck_tconv_x(xt, wc1, bc1, VP)
        else:
            t0 = _tck_tconv_v(h, wc1, bc1, VP)
        to = tin - KT + 1
        p1 = _sc_prop(t0, rows, co, nrm, cnt, VP, NVT)
        p2 = _sc_prop(p1, rows, co, nrm, cnt, VP, NVT)
        tg = _tck_cheb(t0, p1, p2, p["sc%d_cheb_w" % s],
                       p["sc%d_cheb_b" % s], B, to, V)
        wc2, bc2 = _wcat(p, s, 2, C)
        h = _tck_tconv_bn(tg, wc2, bc2, p["sc%d_bn_g" % s],
                          p["sc%d_bn_b" % s])

    return _tck_linear(h, params["lin_w"], params["lin_b"])
